# baseline (speedup 1.0000x reference)
import functools

import jax
import jax.numpy as jnp
from jax import lax
from jax.experimental import pallas as pl
from jax.experimental.pallas import tpu as pltpu
from jax.experimental.pallas import tpu_sc as plsc

B = 4096
NN = 50
NC = 50
NP = NN + NC
D = 64
LANES = 16
PPAD = 64

NW = 32
ROWS_PER_W = B // NW
CB = 16
NCHUNK = ROWS_PER_W // CB
CPAD = 56
ROWPAD = NN + CPAD


def _sc_body(xnp_hbm, xcp_hbm, offp_hbm, tok_hbm, table_hbm, out_hbm,
             xnp_v, xcp_v, offp_v, tok_v, idx_v, asm_v, sem, semo):
    wid = lax.axis_index("s") * 2 + lax.axis_index("c")
    base_row = wid * ROWS_PER_W

    pltpu.sync_copy(tok_hbm, tok_v)
    pltpu.sync_copy(offp_hbm, offp_v)

    @pl.loop(0, NCHUNK)
    def _chunk(ci):
        row0 = base_row + ci * CB
        pltpu.sync_copy(xcp_hbm.at[pl.ds(row0 * PPAD, CB * PPAD)], xcp_v)
        pltpu.sync_copy(xnp_hbm.at[pl.ds(row0 * PPAD, CB * PPAD)], xnp_v)

        @pl.loop(0, CB)
        def _idxrow(b):
            for u in range(PPAD // LANES):
                iv = xcp_v[pl.ds(b * PPAD + LANES * u, LANES)].astype(jnp.int32) \
                    + offp_v[pl.ds(LANES * u, LANES)]
                idx_v[b, pl.ds(LANES * u, LANES)] = iv

        cps = []
        for b in range(CB):
            cps.append(pltpu.async_copy(
                table_hbm.at[idx_v.at[b, pl.ds(0, CPAD)]],
                asm_v.at[b, pl.ds(NN, CPAD)], sem))

        @pl.loop(0, CB)
        def _ncrow(b):
            for u in range(4):
                vx = xnp_v[pl.ds(b * PPAD + LANES * u, LANES)]
                for k in range(LANES if u < 3 else NN - 3 * LANES):
                    p = LANES * u + k
                    s = vx[k]
                    for dd in range(D // LANES):
                        asm_v[b, p, pl.ds(LANES * dd, LANES)] = \
                            tok_v[p, pl.ds(LANES * dd, LANES)] * s

        for cp in cps:
            cp.wait()

        pltpu.async_copy(asm_v.at[:, pl.ds(0, NP)],
                         out_hbm.at[pl.ds(row0, CB)], semo).wait()


@jax.jit
def _tokenize(xnp, xcp, offp, tok, table):
    mesh = plsc.VectorSubcoreMesh(core_axis_name="c", subcore_axis_name="s",
                                  num_cores=2, num_subcores=16)
    f = pl.kernel(
        _sc_body,
        out_type=jax.ShapeDtypeStruct((B, NP, D), jnp.float32),
        mesh=mesh,
        scratch_types=[
            pltpu.VMEM((CB * PPAD,), jnp.float32),
            pltpu.VMEM((CB * PPAD,), jnp.float32),
            pltpu.VMEM((PPAD,), jnp.int32),
            pltpu.VMEM((NN, D), jnp.float32),
            pltpu.VMEM((CB, PPAD), jnp.int32),
            pltpu.VMEM((CB, ROWPAD, D), jnp.float32),
            pltpu.SemaphoreType.DMA,
            pltpu.SemaphoreType.DMA,
        ],
        compiler_params=pltpu.CompilerParams(use_tc_tiling_on_sc=False,
                                             needs_layout_passes=False),
    )
    return f(xnp, xcp, offp, tok, table)


def kernel(x, noncat_tokenizer, cat_table, noncat_idx, cat_idx, cat_offsets):
    xn = x[:, :NN]
    xc = x[:, NN:]
    pad = ((0, 0), (0, PPAD - NN))
    xnp = jnp.pad(xn, pad).reshape(-1)
    xcp = jnp.pad(xc, pad).reshape(-1)
    offp = jnp.pad(cat_offsets.astype(jnp.int32), (0, PPAD - NC))
    return _tokenize(xnp, xcp, offp, noncat_tokenizer, cat_table)


# speedup vs baseline: 1.0043x; 1.0043x over previous
import jax
import jax.numpy as jnp
from jax import lax
from jax.experimental import pallas as pl
from jax.experimental.pallas import tpu as pltpu
from jax.experimental.pallas import tpu_sc as plsc

B = 4096
NN = 50
NC = 50
NP = NN + NC
D = 64
LANES = 16

NW = 32
ROWS_PER_W = B // NW
CB = 16
NCHUNK = ROWS_PER_W // CB
CPAD = 56
ROWPAD = NN + CPAD


def _sc_body(x_hbm, off_hbm, tok_hbm, table_hbm, out_hbm,
             x_v, off_v, tok_v, idx_v, asm_v, sem, semo):
    wid = lax.axis_index("s") * 2 + lax.axis_index("c")
    base_row = wid * ROWS_PER_W
    ivec = lax.iota(jnp.int32, LANES)

    pltpu.sync_copy(tok_hbm, tok_v)
    pltpu.sync_copy(off_hbm, off_v)

    for b in range(CB):
        idx_v[b, pl.ds(48, LANES)] = jnp.zeros((LANES,), jnp.int32)

    @pl.loop(0, NCHUNK)
    def _chunk(ci):
        row0 = base_row + ci * CB
        pltpu.sync_copy(x_hbm.at[pl.ds(row0, CB)], x_v)

        @pl.loop(0, CB)
        def _idxrow(b):
            ib = jnp.full((LANES,), 0, jnp.int32) + b
            for j0 in (0, 16, 32, 34):
                ic = ivec + j0
                codes = plsc.load_gather(x_v, [ib, ic + NN])
                offs = plsc.load_gather(off_v, [ic])
                plsc.store_scatter(idx_v, [ib, ic],
                                   codes.astype(jnp.int32) + offs)

        cps = []
        for b in range(CB):
            cps.append(pltpu.async_copy(
                table_hbm.at[idx_v.at[b, pl.ds(0, CPAD)]],
                asm_v.at[b, pl.ds(NN, CPAD)], sem))

        for p in range(NN):
            tokv = [tok_v[p, pl.ds(LANES * dd, LANES)]
                    for dd in range(D // LANES)]
            ip = jnp.full((LANES,), p, jnp.int32)

            @pl.loop(0, CB, unroll=4)
            def _ncb(b, tokv=tokv, ip=ip, p=p):
                ib = jnp.full((LANES,), 0, jnp.int32) + b
                sv = plsc.load_gather(x_v, [ib, ip])
                for dd in range(D // LANES):
                    asm_v[b, p, pl.ds(LANES * dd, LANES)] = tokv[dd] * sv

        for cp in cps:
            cp.wait()

        pltpu.async_copy(asm_v.at[:, pl.ds(0, NP)],
                         out_hbm.at[pl.ds(row0, CB)], semo).wait()


@jax.jit
def _tokenize(x, off, tok, table):
    mesh = plsc.VectorSubcoreMesh(core_axis_name="c", subcore_axis_name="s",
                                  num_cores=2, num_subcores=16)
    f = pl.kernel(
        _sc_body,
        out_type=jax.ShapeDtypeStruct((B, NP, D), jnp.float32),
        mesh=mesh,
        scratch_types=[
            pltpu.VMEM((CB, NP), jnp.float32),
            pltpu.VMEM((NC,), jnp.int32),
            pltpu.VMEM((NN, D), jnp.float32),
            pltpu.VMEM((CB, 64), jnp.int32),
            pltpu.VMEM((CB, ROWPAD, D), jnp.float32),
            pltpu.SemaphoreType.DMA,
            pltpu.SemaphoreType.DMA,
        ],
        compiler_params=pltpu.CompilerParams(use_tc_tiling_on_sc=False,
                                             needs_layout_passes=False),
    )
    return f(x, off, tok, table)


def kernel(x, noncat_tokenizer, cat_table, noncat_idx, cat_idx, cat_offsets):
    return _tokenize(x, cat_offsets.astype(jnp.int32), noncat_tokenizer,
                     cat_table)
